# baseline (speedup 1.0000x reference)
import functools

import jax
import jax.numpy as jnp
from jax import lax
from jax.experimental import pallas as pl
from jax.experimental.pallas import tpu as pltpu
from jax.experimental.pallas import tpu_sc as plsc

R, C = 128, 32768
NC, NS, L = 2, 16, 16
NW = NC * NS
TJ = C // 128

RS = 32
TS = RS // 8
G = NW // TS
GJ = TJ // G

RT = R - RS
MT = RT // 8


_mesh = plsc.VectorSubcoreMesh(
    core_axis_name="c", subcore_axis_name="s", num_cores=NC, num_subcores=NS)


@functools.partial(
    pl.kernel,
    out_type=(jax.ShapeDtypeStruct((NW, 16 * L), jnp.float32),
              jax.ShapeDtypeStruct((NW, 16 * L), jnp.int32)),
    mesh=_mesh,
    scratch_types=[
        pltpu.VMEM((GJ * 1024,), jnp.float32),
        pltpu.VMEM((16 * L,), jnp.float32),
        pltpu.VMEM((16 * L,), jnp.int32),
        pltpu.SemaphoreType.DMA,
    ],
    compiler_params=pltpu.CompilerParams(needs_layout_passes=False),
)
def _argmax_sc(x_hbm, outv_hbm, outi_hbm, buf, vals_v, idxs_v, sem0):
    c = lax.axis_index("c")
    s = lax.axis_index("s")
    unit = c * NS + s
    t = lax.div(unit, G)
    g = lax.rem(unit, G)
    base = (t * TJ + g * GJ) * 1024

    cp = pltpu.async_copy(x_hbm.at[pl.ds(base, GJ * 1024)], buf, sem0)

    iota = lax.iota(jnp.int32, L)
    ninf = jnp.full((L,), -jnp.inf, jnp.float32)
    cid0 = g * GJ * 8
    cp.wait()

    def make_body(kks):
        def body(jj, carry):
            a = list(carry)
            off = jj * 1024
            cid_base = cid0 + jj * 8
            for ki, kk in enumerate(kks):
                for cc in range(8):
                    p = cc & 1
                    slot = 4 * ki + 2 * p
                    b, i = a[slot], a[slot + 1]
                    x = buf[pl.ds(off + kk * 128 + cc * 16, 16)]
                    m = x > b
                    cid = jnp.full((L,), cid_base + cc, jnp.int32)
                    a[slot] = jnp.where(m, x, b)
                    a[slot + 1] = jnp.where(m, cid, i)
            return tuple(a)
        return body

    zero = jnp.zeros((L,), jnp.int32)
    for kks in ((0, 1, 2, 3), (4, 5, 6, 7)):
        carry = (ninf, zero) * 8
        carry = lax.fori_loop(0, GJ, make_body(kks), carry)
        for ki, kk in enumerate(kks):
            for p in range(2):
                b = carry[4 * ki + 2 * p]
                i = carry[4 * ki + 2 * p + 1]
                slot = (kk * 2 + p) * L
                vals_v[pl.ds(slot, L)] = b
                idxs_v[pl.ds(slot, L)] = i * 16 + iota
    pltpu.sync_copy(vals_v, outv_hbm.at[unit])
    pltpu.sync_copy(idxs_v, outi_hbm.at[unit])


NV = C // 128
TCA = 4


def _argmax_tc_body(x_ref, o_ref):
    best = [x_ref[:, a * 128:(a + 1) * 128] for a in range(TCA)]
    bidx = [jnp.full((8, 128), a, jnp.int32) for a in range(TCA)]
    for j in range(1, NV // TCA):
        for a in range(TCA):
            jj = j * TCA + a
            x = x_ref[:, jj * 128:(jj + 1) * 128]
            m = x > best[a]
            best[a] = jnp.where(m, x, best[a])
            bidx[a] = jnp.where(m, jnp.full((8, 128), jj, jnp.int32),
                                bidx[a])
    b, i = best[0], bidx[0]
    for a in range(1, TCA):
        take = (best[a] > b) | ((best[a] == b) & (bidx[a] < i))
        b = jnp.where(take, best[a], b)
        i = jnp.where(take, bidx[a], i)
    mx = jnp.max(b, axis=1, keepdims=True)
    cols = i * 128 + lax.broadcasted_iota(jnp.int32, (8, 128), 1)
    o_ref[...] = jnp.min(jnp.where(b == mx, cols, C), axis=1,
                         keepdims=True)


_argmax_tc = pl.pallas_call(
    _argmax_tc_body,
    grid=(MT,),
    in_specs=[pl.BlockSpec((8, C), lambda m: (TS + m, 0))],
    out_specs=pl.BlockSpec((8, 1), lambda m: (m, 0)),
    out_shape=jax.ShapeDtypeStruct((RT, 1), jnp.int32),
    compiler_params=pltpu.CompilerParams(
        dimension_semantics=("arbitrary",),
        vmem_limit_bytes=57 * 1024 * 1024),
)


def kernel(tensor):
    x1 = (tensor.reshape(R // 8, 8, TJ, 128)
          .transpose(0, 2, 1, 3)
          .reshape(R * C))
    outv, outi = _argmax_sc(x1)
    idx_tc = _argmax_tc(tensor)
    v = outv.reshape(TS, G, 8, 2 * L)
    i = outi.reshape(TS, G, 8, 2 * L)
    mx = jnp.max(v, axis=(1, 3), keepdims=True)
    cand = jnp.where(v == mx, i, C)
    idx_sc = jnp.min(cand, axis=(1, 3)).reshape(RS, 1)
    return jnp.concatenate([idx_sc, idx_tc], axis=0)


# speedup vs baseline: 1.0456x; 1.0456x over previous
import functools

import jax
import jax.numpy as jnp
from jax import lax
from jax.experimental import pallas as pl
from jax.experimental.pallas import tpu as pltpu
from jax.experimental.pallas import tpu_sc as plsc

R, C = 128, 32768
NC, NS, L = 2, 16, 16
NW = NC * NS
TJ = C // 128

RS = 32
TS = RS // 8
G = NW // TS
GJ = TJ // G

RT = R - RS
MT = RT // 8


def _merge(b0, i0, b1, i1):
    take = (b1 > b0) | ((b1 == b0) & (i1 < i0))
    return jnp.where(take, b1, b0), jnp.where(take, i1, i0)


def _butterfly(b, i, iota, vscr, iscr):
    for s in (1, 2, 4, 8):
        vscr[...] = b
        iscr[...] = i
        perm = iota ^ s
        b_p = plsc.load_gather(vscr, [perm])
        i_p = plsc.load_gather(iscr, [perm])
        b, i = _merge(b, i, b_p, i_p)
    return b, i


_mesh = plsc.VectorSubcoreMesh(
    core_axis_name="c", subcore_axis_name="s", num_cores=NC, num_subcores=NS)


@functools.partial(
    pl.kernel,
    out_type=(jax.ShapeDtypeStruct((NW, L), jnp.float32),
              jax.ShapeDtypeStruct((NW, L), jnp.int32)),
    mesh=_mesh,
    scratch_types=[
        pltpu.VMEM((GJ * 1024,), jnp.float32),
        pltpu.VMEM((L,), jnp.float32),
        pltpu.VMEM((L,), jnp.int32),
        pltpu.VMEM((L,), jnp.float32),
        pltpu.VMEM((L,), jnp.int32),
        pltpu.SemaphoreType.DMA,
    ],
    compiler_params=pltpu.CompilerParams(needs_layout_passes=False),
)
def _argmax_sc(x_hbm, outv_hbm, outi_hbm, buf, vscr, iscr,
               vals_v, idxs_v, sem0):
    c = lax.axis_index("c")
    s = lax.axis_index("s")
    unit = c * NS + s
    t = lax.div(unit, G)
    g = lax.rem(unit, G)
    base = (t * TJ + g * GJ) * 1024

    cp = pltpu.async_copy(x_hbm.at[pl.ds(base, GJ * 1024)], buf, sem0)

    iota = lax.iota(jnp.int32, L)
    ninf = jnp.full((L,), -jnp.inf, jnp.float32)
    cid0 = g * GJ * 8
    cp.wait()

    def make_body(kks):
        def body(jj, carry):
            a = list(carry)
            off = jj * 1024
            cid_base = cid0 + jj * 8
            for ki, kk in enumerate(kks):
                for cc in range(8):
                    p = cc & 1
                    slot = 4 * ki + 2 * p
                    b, i = a[slot], a[slot + 1]
                    x = buf[pl.ds(off + kk * 128 + cc * 16, 16)]
                    m = x > b
                    cid = jnp.full((L,), cid_base + cc, jnp.int32)
                    a[slot] = jnp.where(m, x, b)
                    a[slot + 1] = jnp.where(m, cid, i)
            return tuple(a)
        return body

    vals = jnp.zeros((L,), jnp.float32)
    idxs = jnp.zeros((L,), jnp.int32)
    zero = jnp.zeros((L,), jnp.int32)
    for kks in ((0, 1, 2, 3), (4, 5, 6, 7)):
        carry = (ninf, zero) * 8
        carry = lax.fori_loop(0, GJ, make_body(kks), carry)
        for ki, kk in enumerate(kks):
            b0, i0 = carry[4 * ki], carry[4 * ki + 1]
            b1, i1 = carry[4 * ki + 2], carry[4 * ki + 3]
            b, i = _merge(b0, i0 * 16 + iota, b1, i1 * 16 + iota)
            b, i = _butterfly(b, i, iota, vscr, iscr)
            sel = iota == kk
            vals = jnp.where(sel, b, vals)
            idxs = jnp.where(sel, i, idxs)

    vals_v[...] = vals
    idxs_v[...] = idxs
    pltpu.sync_copy(vals_v, outv_hbm.at[unit])
    pltpu.sync_copy(idxs_v, outi_hbm.at[unit])


NV = C // 128
TCA = 4


def _argmax_tc_body(x_ref, o_ref):
    best = [x_ref[:, a * 128:(a + 1) * 128] for a in range(TCA)]
    bidx = [jnp.full((8, 128), a, jnp.int32) for a in range(TCA)]
    for j in range(1, NV // TCA):
        for a in range(TCA):
            jj = j * TCA + a
            x = x_ref[:, jj * 128:(jj + 1) * 128]
            m = x > best[a]
            best[a] = jnp.where(m, x, best[a])
            bidx[a] = jnp.where(m, jnp.full((8, 128), jj, jnp.int32),
                                bidx[a])
    b, i = best[0], bidx[0]
    for a in range(1, TCA):
        take = (best[a] > b) | ((best[a] == b) & (bidx[a] < i))
        b = jnp.where(take, best[a], b)
        i = jnp.where(take, bidx[a], i)
    mx = jnp.max(b, axis=1, keepdims=True)
    cols = i * 128 + lax.broadcasted_iota(jnp.int32, (8, 128), 1)
    o_ref[...] = jnp.min(jnp.where(b == mx, cols, C), axis=1,
                         keepdims=True)


_argmax_tc = pl.pallas_call(
    _argmax_tc_body,
    grid=(MT,),
    in_specs=[pl.BlockSpec((8, C), lambda m: (TS + m, 0))],
    out_specs=pl.BlockSpec((8, 1), lambda m: (m, 0)),
    out_shape=jax.ShapeDtypeStruct((RT, 1), jnp.int32),
    compiler_params=pltpu.CompilerParams(
        dimension_semantics=("arbitrary",),
        vmem_limit_bytes=57 * 1024 * 1024),
)


def kernel(tensor):
    x1 = (tensor.reshape(R // 8, 8, TJ, 128)
          .transpose(0, 2, 1, 3)
          .reshape(R * C))
    outv, outi = _argmax_sc(x1)
    idx_tc = _argmax_tc(tensor)
    v = outv.reshape(TS, G, L)[:, :, :8]
    i = outi.reshape(TS, G, L)[:, :, :8]
    mx = jnp.max(v, axis=1, keepdims=True)
    cand = jnp.where(v == mx, i, C)
    idx_sc = jnp.min(cand, axis=1).reshape(RS, 1)
    return jnp.concatenate([idx_sc, idx_tc], axis=0)
